# Initial kernel scaffold; baseline (speedup 1.0000x reference)
import jax
import jax.numpy as jnp
from jax.experimental import pallas as pl


def kernel(node_features, W1, b1, W2, b2, neighbor_indices, neighbor_weights):
    raise NotImplementedError("write your pallas kernel here")


# speedup vs baseline: 2.5659x; 2.5659x over previous
import jax
import jax.numpy as jnp
from jax.experimental import pallas as pl
from jax.experimental.pallas import tpu as pltpu

_N_LAT, _N_LON, _N_MESH, _N_NEI = 64, 128, 128, 4
_IN_DIM, _HID, _OUT_CH = 256, 256, 78
_BATCH = 32
_N_GRID = _N_LAT * _N_LON
_GB = 1024
_MB = 1024


def _mlp_body(nf_ref, w1_ref, b1_ref, w2_ref, b2_ref, out_ref):
    h = jnp.dot(nf_ref[...], w1_ref[...],
                preferred_element_type=jnp.float32,
                precision=jax.lax.Precision.HIGHEST)
    h = jnp.maximum(h + b1_ref[...], 0.0)
    o = jnp.dot(h, w2_ref[...],
                preferred_element_type=jnp.float32,
                precision=jax.lax.Precision.HIGHEST)
    out_ref[...] = (o + b2_ref[...]).astype(jnp.bfloat16)


def _interp_body(idx_ref, wts_ref, mesh_ref, out_ref, a_ref):
    b = pl.program_id(1)

    @pl.when(b == 0)
    def _build_a():
        iota = jax.lax.broadcasted_iota(jnp.int32, (_GB, _N_MESH), 1)
        acc = jnp.zeros((_GB, _N_MESH), jnp.float32)
        for k in range(_N_NEI):
            acc = acc + jnp.where(idx_ref[:, k:k + 1] == iota,
                                  wts_ref[:, k:k + 1], 0.0)
        a_ref[...] = acc.astype(jnp.bfloat16)

    out_ref[0] = jax.lax.dot_general(
        a_ref[...], mesh_ref[0],
        (((1,), (0,)), ((), ())),
        preferred_element_type=jnp.float32)


def kernel(node_features, W1, b1, W2, b2, neighbor_indices, neighbor_weights):
    nf2 = node_features.reshape(_BATCH * _N_MESH, _IN_DIM)

    mesh = pl.pallas_call(
        _mlp_body,
        grid=(_BATCH * _N_MESH // _MB,),
        in_specs=[
            pl.BlockSpec((_MB, _IN_DIM), lambda i: (i, 0)),
            pl.BlockSpec((_IN_DIM, _HID), lambda i: (0, 0)),
            pl.BlockSpec((1, _HID), lambda i: (0, 0)),
            pl.BlockSpec((_HID, _OUT_CH), lambda i: (0, 0)),
            pl.BlockSpec((1, _OUT_CH), lambda i: (0, 0)),
        ],
        out_specs=pl.BlockSpec((_MB, _OUT_CH), lambda i: (i, 0)),
        out_shape=jax.ShapeDtypeStruct((_BATCH * _N_MESH, _OUT_CH),
                                       jnp.bfloat16),
        compiler_params=pltpu.CompilerParams(
            dimension_semantics=("parallel",)),
    )(nf2, W1, b1.reshape(1, _HID), W2, b2.reshape(1, _OUT_CH))

    mesh = mesh.reshape(_BATCH, _N_MESH, _OUT_CH)

    out = pl.pallas_call(
        _interp_body,
        grid=(_N_GRID // _GB, _BATCH),
        in_specs=[
            pl.BlockSpec((_GB, _N_NEI), lambda g, b: (g, 0)),
            pl.BlockSpec((_GB, _N_NEI), lambda g, b: (g, 0)),
            pl.BlockSpec((1, _N_MESH, _OUT_CH), lambda g, b: (b, 0, 0)),
        ],
        out_specs=pl.BlockSpec((1, _GB, _OUT_CH), lambda g, b: (b, g, 0)),
        out_shape=jax.ShapeDtypeStruct((_BATCH, _N_GRID, _OUT_CH),
                                       jnp.float32),
        scratch_shapes=[pltpu.VMEM((_GB, _N_MESH), jnp.bfloat16)],
        compiler_params=pltpu.CompilerParams(
            dimension_semantics=("parallel", "arbitrary")),
    )(neighbor_indices, neighbor_weights, mesh)

    return out.reshape(_BATCH, _N_LAT, _N_LON, _OUT_CH)
